# Initial kernel scaffold; baseline (speedup 1.0000x reference)
import jax
import jax.numpy as jnp
from jax.experimental import pallas as pl


def kernel(z, edge_index, batch, z_table, Wl0, bl0, Wr0, Wl1, bl1, Wr1, Wl2, bl2, Wr2, lin1_W, lin1_b, lin2_W, lin2_b):
    raise NotImplementedError("write your pallas kernel here")


# speedup vs baseline: 7.0505x; 7.0505x over previous
import jax
import jax.numpy as jnp
from jax import lax
from jax.experimental import pallas as pl
from jax.experimental.pallas import tpu as pltpu
from jax.experimental.pallas import tpu_sc as plsc

NC = 2
NS = 16
L = 16
NW = NC * NS
H = 128
CW = 16
CH = 100
ZR = 25

_MESH = plsc.VectorSubcoreMesh(
    core_axis_name="c", subcore_axis_name="s", num_cores=NC, num_subcores=NS)
_PARAMS = pltpu.CompilerParams(use_tc_tiling_on_sc=False,
                               needs_layout_passes=False)


def _embed_body(z_hbm, tab_hbm, out_hbm, idx_v, rows_v, sem):
    cid = lax.axis_index("c")
    sid = lax.axis_index("s")
    wid = sid * NC + cid
    n = z_hbm.shape[0]
    nch = n // L
    base = nch // NW
    rem = nch - base * NW
    cnt = base + jnp.where(wid < rem, 1, 0)

    @pl.loop(0, cnt)
    def _(j):
        b = (j * NW + wid) * L
        pltpu.sync_copy(z_hbm.at[pl.ds(b, L)], idx_v)
        pltpu.async_copy(tab_hbm.at[idx_v], rows_v, sem).wait()
        pltpu.sync_copy(rows_v, out_hbm.at[pl.ds(b, L)])


def _embed(z, tab):
    n = z.shape[0]
    return pl.kernel(
        _embed_body,
        out_type=jax.ShapeDtypeStruct((n, H), jnp.float32),
        mesh=_MESH,
        compiler_params=_PARAMS,
        scratch_types=[
            pltpu.VMEM((L,), jnp.int32),
            pltpu.VMEM((L, H), jnp.float32),
            pltpu.SemaphoreType.DMA,
        ],
    )(z, tab)


def _deg_body(dst_hbm, out_hbm, dst_v, ones_v, zb_v, cnt_s):
    cid = lax.axis_index("c")
    sid = lax.axis_index("s")
    wid = sid * NC + cid
    n = out_hbm.shape[1]
    nche = dst_hbm.shape[1]
    rows_per_tile = n // NS

    pltpu.sync_copy(dst_hbm.at[wid], dst_v)

    @pl.loop(0, CH)
    def _(r):
        ones_v[r, pl.ds(0, CW)] = jnp.ones((CW,), jnp.float32)

    @pl.loop(0, ZR)
    def _(r):
        zb_v[r, pl.ds(0, CW)] = jnp.zeros((CW,), jnp.float32)

    @pl.loop(0, rows_per_tile // ZR)
    def _(t):
        pltpu.sync_copy(zb_v, cnt_s.at[pl.ds(sid * rows_per_tile + t * ZR, ZR)])
    plsc.subcore_barrier()

    @pl.loop(0, nche)
    def _(j):
        pltpu.sync_copy(ones_v, cnt_s.at[dst_v.at[j]], add=True)

    plsc.subcore_barrier()

    @pl.loop(0, rows_per_tile // ZR)
    def _(t):
        r0 = sid * rows_per_tile + t * ZR
        pltpu.sync_copy(cnt_s.at[pl.ds(r0, ZR)], zb_v)
        pltpu.sync_copy(zb_v, out_hbm.at[cid, pl.ds(r0, ZR)])


def _deg(dst, n):
    return pl.kernel(
        _deg_body,
        out_type=jax.ShapeDtypeStruct((NC, n, CW), jnp.float32),
        mesh=_MESH,
        compiler_params=_PARAMS,
        scratch_types=[
            pltpu.VMEM(dst.shape[1:], jnp.int32),
            pltpu.VMEM((CH, CW), jnp.float32),
            pltpu.VMEM((ZR, CW), jnp.float32),
            pltpu.VMEM_SHARED((n, CW), jnp.float32),
        ],
    )(dst)


def _agg_body(x_hbm, src_hbm, dst_hbm, out_hbm,
              src_v, dst_v, rows_v, zb_v, agg_s, sem0, sem1):
    cid = lax.axis_index("c")
    sid = lax.axis_index("s")
    wid = sid * NC + cid
    n = x_hbm.shape[0]
    nche = src_hbm.shape[1]
    rows_per_tile = n // NS

    pltpu.sync_copy(src_hbm.at[wid], src_v)
    pltpu.sync_copy(dst_hbm.at[wid], dst_v)

    sems = (sem0, sem1)

    def start(j, b):
        pltpu.async_copy(x_hbm.at[src_v.at[j]], rows_v.at[b], sems[b])

    def wait(b):
        pltpu.make_async_copy(x_hbm.at[pl.ds(0, CH)], rows_v.at[b],
                              sems[b]).wait()

    start(0, 0)
    start(1, 1)

    @pl.loop(0, ZR)
    def _(r):
        for cc in range(H // L):
            zb_v[r, pl.ds(cc * L, L)] = jnp.zeros((L,), jnp.float32)

    @pl.loop(0, rows_per_tile // ZR)
    def _(t):
        pltpu.sync_copy(zb_v, agg_s.at[pl.ds(sid * rows_per_tile + t * ZR, ZR)])
    plsc.subcore_barrier()

    @pl.loop(0, nche, step=2)
    def _(j):
        for b in range(2):
            wait(b)
            pltpu.sync_copy(rows_v.at[b], agg_s.at[dst_v.at[j + b]],
                            add=True)
            nxt = j + 2 + b

            @pl.when(nxt < nche)
            def _():
                start(nxt, b)

    plsc.subcore_barrier()
    @pl.loop(0, rows_per_tile // ZR)
    def _(t):
        r0 = sid * rows_per_tile + t * ZR
        pltpu.sync_copy(agg_s.at[pl.ds(r0, ZR)], zb_v)
        pltpu.sync_copy(zb_v, out_hbm.at[cid, pl.ds(r0, ZR)])


def _agg(x, src, dst):
    n = x.shape[0]
    return pl.kernel(
        _agg_body,
        out_type=jax.ShapeDtypeStruct((NC, n, H), jnp.float32),
        mesh=_MESH,
        compiler_params=_PARAMS,
        scratch_types=[
            pltpu.VMEM(src.shape[1:], jnp.int32),
            pltpu.VMEM(dst.shape[1:], jnp.int32),
            pltpu.VMEM((2, CH, H), jnp.float32),
            pltpu.VMEM((ZR, H), jnp.float32),
            pltpu.VMEM_SHARED((n, H), jnp.float32),
            pltpu.SemaphoreType.DMA,
            pltpu.SemaphoreType.DMA,
        ],
    )(x, src, dst)


def _pool_body(batch_hbm, x_hbm, xs_hbm, xd_hbm,
               bch_v, ci_v, idxa_v, idxb_v, rows_v, ci_s, sem):
    cid = lax.axis_index("c")
    sid = lax.axis_index("s")
    wid = sid * NC + cid
    n = batch_hbm.shape[0]
    g = xs_hbm.shape[0]
    lanes = lax.iota(jnp.int32, L)

    @pl.when(sid == 0)
    def _():
        @pl.loop(0, g // L)
        def _(i):
            ci_v[pl.ds(i * L, L)] = jnp.zeros((L,), jnp.int32)

        shift_idx = jnp.maximum(lanes - 1, 0)[:, None]
        dn = lax.GatherDimensionNumbers(
            offset_dims=(), collapsed_slice_dims=(0,), start_index_map=(0,))

        @pl.loop(0, n // L, init_carry=(jnp.int32(0), jnp.int32(-1)))
        def _(c, carry):
            cntp, prev = carry
            pltpu.sync_copy(batch_hbm.at[pl.ds(c * L, L)], bch_v)
            bv = bch_v[...]
            bshift = lax.gather(bv, shift_idx, dn, slice_sizes=(1,),
                                mode=lax.GatherScatterMode.PROMISE_IN_BOUNDS)
            prevv = jnp.where(lanes == 0, prev, bshift)
            m = bv != prevv
            mi = jnp.where(m, 1, 0).astype(jnp.int32)
            pos = cntp + plsc.cumsum(mi) - 1
            plsc.store_scatter(ci_v, [pos], c * L + lanes, mask=m)
            newcnt = cntp + jnp.sum(mi)
            newprev = jnp.max(jnp.where(lanes == L - 1, bv,
                                        jnp.int32(-2147483648)))
            return (newcnt, newprev)

        pltpu.sync_copy(ci_v, ci_s)

    plsc.subcore_barrier()

    g0 = wid * L
    pltpu.sync_copy(ci_s.at[pl.ds(g0, L)], bch_v)
    civ = jnp.minimum(bch_v[...], n - 2)
    idxa_v[...] = civ
    idxb_v[...] = civ + 1
    pltpu.async_copy(x_hbm.at[idxa_v], rows_v, sem).wait()
    pltpu.sync_copy(rows_v, xs_hbm.at[pl.ds(g0, L)])
    pltpu.async_copy(x_hbm.at[idxb_v], rows_v, sem).wait()
    pltpu.sync_copy(rows_v, xd_hbm.at[pl.ds(g0, L)])


def _pool(batch, x, g):
    sds = jax.ShapeDtypeStruct((g, H), jnp.float32)
    return pl.kernel(
        _pool_body,
        out_type=(sds, sds),
        mesh=_MESH,
        compiler_params=_PARAMS,
        scratch_types=[
            pltpu.VMEM((L,), jnp.int32),
            pltpu.VMEM((g,), jnp.int32),
            pltpu.VMEM((L,), jnp.int32),
            pltpu.VMEM((L,), jnp.int32),
            pltpu.VMEM((L, H), jnp.float32),
            pltpu.VMEM_SHARED((g,), jnp.int32),
            pltpu.SemaphoreType.DMA,
        ],
    )(batch, x)


def _tc_layer(x, p0, p1, c0, c1, Wl, bl, Wr, relu):
    n = x.shape[0]
    bn = 2000

    def body(x_ref, p0_ref, p1_ref, c0_ref, c1_ref, wl_ref, wr_ref, bl_ref,
             o_ref):
        agg = p0_ref[...] + p1_ref[...]
        cnt = c0_ref[:, :1] + c1_ref[:, :1]
        mean = agg / jnp.maximum(cnt, 1.0)
        y = jnp.dot(mean, wl_ref[...], preferred_element_type=jnp.float32)
        y = y + jnp.dot(x_ref[...], wr_ref[...],
                        preferred_element_type=jnp.float32)
        y = y + bl_ref[...]
        if relu:
            y = jnp.maximum(y, 0.0)
        o_ref[...] = y

    return pl.pallas_call(
        body,
        grid=(n // bn,),
        in_specs=[
            pl.BlockSpec((bn, H), lambda i: (i, 0)),
            pl.BlockSpec((bn, H), lambda i: (i, 0)),
            pl.BlockSpec((bn, H), lambda i: (i, 0)),
            pl.BlockSpec((bn, CW), lambda i: (i, 0)),
            pl.BlockSpec((bn, CW), lambda i: (i, 0)),
            pl.BlockSpec((H, H), lambda i: (0, 0)),
            pl.BlockSpec((H, H), lambda i: (0, 0)),
            pl.BlockSpec((1, H), lambda i: (0, 0)),
        ],
        out_specs=pl.BlockSpec((bn, H), lambda i: (i, 0)),
        out_shape=jax.ShapeDtypeStruct((n, H), jnp.float32),
    )(x, p0, p1, c0, c1, Wl, Wr, bl.reshape(1, H))


def _tc_head(xs, xd, W1, b1, W2, b2):
    g = xs.shape[0]

    def body(xs_ref, xd_ref, w1_ref, b1_ref, w2_ref, b2_ref, o_ref):
        h = xs_ref[...] * xd_ref[...]
        h1 = jnp.dot(h, w1_ref[...], preferred_element_type=jnp.float32)
        h1 = jnp.maximum(h1 + b1_ref[...], 0.0)
        o_ref[...] = jnp.dot(h1, w2_ref[...],
                             preferred_element_type=jnp.float32) + b2_ref[...]

    return pl.pallas_call(
        body,
        out_shape=jax.ShapeDtypeStruct((g, 1), jnp.float32),
    )(xs, xd, W1, b1.reshape(1, H), W2, b2.reshape(1, 1))


def kernel(z, edge_index, batch, z_table,
           Wl0, bl0, Wr0, Wl1, bl1, Wr1, Wl2, bl2, Wr2,
           lin1_W, lin1_b, lin2_W, lin2_b):
    n = z.shape[0]
    e = edge_index.shape[1]
    g = 512

    z = z.astype(jnp.int32)
    batch = batch.astype(jnp.int32)
    src = edge_index[0].astype(jnp.int32).reshape(NW, e // (NW * CH), CH)
    dst = edge_index[1].astype(jnp.int32).reshape(NW, e // (NW * CH), CH)

    cp = _deg(dst, n)
    c0, c1 = cp[0], cp[1]
    x = _embed(z, z_table)
    for Wl, bl, Wr, relu in ((Wl0, bl0, Wr0, True),
                             (Wl1, bl1, Wr1, True),
                             (Wl2, bl2, Wr2, False)):
        part = _agg(x, src, dst)
        x = _tc_layer(x, part[0], part[1], c0, c1, Wl, bl, Wr, relu)

    xs, xd = _pool(batch, x, g)
    return _tc_head(xs, xd, lin1_W, lin1_b, lin2_W, lin2_b)
